# Initial kernel scaffold; baseline (speedup 1.0000x reference)
import jax
import jax.numpy as jnp
from jax.experimental import pallas as pl


def kernel(preds, targets):
    raise NotImplementedError("write your pallas kernel here")


# speedup vs baseline: 10.4299x; 10.4299x over previous
import functools

import jax
import jax.numpy as jnp
from jax import lax
from jax.experimental import pallas as pl
from jax.experimental.pallas import tpu as pltpu
from jax.experimental.pallas import tpu_sc as plsc

C = 4096
L = 16
NCHUNK = C // L
NC = 2
NS = 16
NW = NC * NS
RPW = C // NW
LABEL_SMOOTH = 0.1
EPS = LABEL_SMOOTH / (C - 1)
HI = 1.0 - LABEL_SMOOTH


def _sc_body(preds_hbm, tgt_hbm, stats_hbm, cval_hbm, cidx_hbm,
             rowbuf, tgtbuf, stats_v, cval_v, cidx_v):
    wid = lax.axis_index("s") * NC + lax.axis_index("c")
    base = wid * RPW
    pltpu.sync_copy(tgt_hbm.at[pl.ds(base, RPW)], tgtbuf)

    iota = lax.iota(jnp.int32, L)
    zf = jnp.zeros((L,), jnp.float32)
    zi = jnp.zeros((L,), jnp.int32)
    ninf = jnp.full((L,), -jnp.inf, jnp.float32)

    def row_step(j, _):
        row = base + j
        pltpu.sync_copy(preds_hbm.at[pl.ds(row * C, C)], rowbuf)

        def p1(c, carry):
            s, r0v, r1v, r2v, r3v, r0i, r1i, r2i, r3i = carry
            v = rowbuf[pl.ds(c * L, L)]
            cols = iota + c * L
            s = s + v
            w0 = v > r0v
            w1 = v > r1v
            w2 = v > r2v
            w3 = v > r3v
            n0v = jnp.where(w0, v, r0v)
            n0i = jnp.where(w0, cols, r0i)
            n1v = jnp.where(w0, r0v, jnp.where(w1, v, r1v))
            n1i = jnp.where(w0, r0i, jnp.where(w1, cols, r1i))
            n2v = jnp.where(w1, r1v, jnp.where(w2, v, r2v))
            n2i = jnp.where(w1, r1i, jnp.where(w2, cols, r2i))
            n3v = jnp.where(w2, r2v, jnp.where(w3, v, r3v))
            n3i = jnp.where(w2, r2i, jnp.where(w3, cols, r3i))
            return (s, n0v, n1v, n2v, n3v, n0i, n1i, n2i, n3i)

        s, r0v, r1v, r2v, r3v, r0i, r1i, r2i, r3i = lax.fori_loop(
            0, NCHUNK, p1, (zf, ninf, ninf, ninf, ninf, zi, zi, zi, zi))
        m_v = r0v

        def p2(c, e):
            v = rowbuf[pl.ds(c * L, L)]
            return e + jnp.exp(v - m_v)

        e_v = lax.fori_loop(0, NCHUNK, p2, zf)

        tj = plsc.load_gather(tgtbuf, [jnp.full((L,), j, jnp.int32)])
        ptv = plsc.load_gather(rowbuf, [tj])

        sb = j * 64
        stats_v[pl.ds(sb, L)] = m_v
        stats_v[pl.ds(sb + 16, L)] = s
        stats_v[pl.ds(sb + 32, L)] = e_v
        stats_v[pl.ds(sb + 48, L)] = ptv
        cval_v[pl.ds(sb, L)] = r0v
        cval_v[pl.ds(sb + 16, L)] = r1v
        cval_v[pl.ds(sb + 32, L)] = r2v
        cval_v[pl.ds(sb + 48, L)] = r3v
        cidx_v[pl.ds(sb, L)] = r0i
        cidx_v[pl.ds(sb + 16, L)] = r1i
        cidx_v[pl.ds(sb + 32, L)] = r2i
        cidx_v[pl.ds(sb + 48, L)] = r3i
        return 0

    lax.fori_loop(0, RPW, row_step, 0)

    pltpu.sync_copy(stats_v, stats_hbm.at[pl.ds(base * 64, RPW * 64)])
    pltpu.sync_copy(cval_v, cval_hbm.at[pl.ds(base * 64, RPW * 64)])
    pltpu.sync_copy(cidx_v, cidx_hbm.at[pl.ds(base * 64, RPW * 64)])


def _fin_body(stats_ref, cval_ref, cidx_ref, tgt_ref, out_ref):
    i = pl.program_id(0)
    R = stats_ref.shape[0]
    stats = stats_ref[...]
    m_v = stats[:, 0:16]
    sum_v = stats[:, 16:32]
    e_v = stats[:, 32:48]
    pt = jnp.max(stats[:, 48:64], axis=1)

    M = jnp.max(m_v, axis=1)
    S = jnp.sum(e_v * jnp.exp(m_v - M[:, None]), axis=1)
    lse = M + jnp.log(S)
    rowsum = jnp.sum(sum_v, axis=1)
    full = EPS * (rowsum - C * lse) + (HI - EPS) * (pt - lse)

    cval = cval_ref[...]
    cidx = cidx_ref[...]
    alive = jnp.ones(cval.shape, jnp.bool_)
    tv = []
    ti = []
    for _ in range(4):
        mv = jnp.where(alive, cval, -jnp.inf)
        cur = jnp.max(mv, axis=1)
        cand = mv == cur[:, None]
        curi = jnp.min(jnp.where(cand, cidx, C), axis=1)
        tv.append(cur)
        ti.append(curi)
        alive = alive & ~(cand & (cidx == curi[:, None]))

    rows = i * R + lax.broadcasted_iota(jnp.int32, (R,), 0)
    in0 = ti[0] == rows
    in1 = ti[1] == rows
    in2 = ti[2] == rows
    tgt = tgt_ref[:, 0]

    def term(v, idx):
        w = jnp.where(idx == tgt, HI, EPS)
        return w * (v - lse)

    sk0 = jnp.where(in0, term(tv[1], ti[1]), term(tv[0], ti[0]))
    sk1 = jnp.where(in0 | in1, term(tv[2], ti[2]), term(tv[1], ti[1]))
    sk2 = jnp.where(in0 | in1 | in2, term(tv[3], ti[3]), term(tv[2], ti[2]))
    skipped = sk0 + sk1 + sk2

    loss = -(full - skipped)
    part = jnp.reshape(jnp.sum(loss) * (1.0 / C), (1, 1))

    @pl.when(i == 0)
    def _():
        out_ref[...] = jnp.zeros((1, 1), jnp.float32)

    out_ref[...] += part


def _sc_call(preds_flat, targets):
    mesh = plsc.VectorSubcoreMesh(core_axis_name="c", subcore_axis_name="s",
                                  num_cores=NC, num_subcores=NS)
    f = functools.partial(
        pl.kernel,
        mesh=mesh,
        out_type=[
            jax.ShapeDtypeStruct((C * 64,), jnp.float32),
            jax.ShapeDtypeStruct((C * 64,), jnp.float32),
            jax.ShapeDtypeStruct((C * 64,), jnp.int32),
        ],
        scratch_types=[
            pltpu.VMEM((C,), jnp.float32),
            pltpu.VMEM((RPW,), jnp.int32),
            pltpu.VMEM((RPW * 64,), jnp.float32),
            pltpu.VMEM((RPW * 64,), jnp.float32),
            pltpu.VMEM((RPW * 64,), jnp.int32),
        ],
        compiler_params=pltpu.CompilerParams(needs_layout_passes=False),
    )(_sc_body)
    return f(preds_flat, targets)


def kernel(preds, targets):
    preds_flat = preds.reshape(-1)
    tgt = targets.astype(jnp.int32)
    stats, cval, cidx = _sc_call(preds_flat, tgt)

    R = 512
    out = pl.pallas_call(
        _fin_body,
        grid=(C // R,),
        in_specs=[
            pl.BlockSpec((R, 64), lambda i: (i, 0)),
            pl.BlockSpec((R, 64), lambda i: (i, 0)),
            pl.BlockSpec((R, 64), lambda i: (i, 0)),
            pl.BlockSpec((R, 1), lambda i: (i, 0)),
        ],
        out_specs=pl.BlockSpec((1, 1), lambda i: (0, 0)),
        out_shape=jax.ShapeDtypeStruct((1, 1), jnp.float32),
    )(stats.reshape(C, 64), cval.reshape(C, 64), cidx.reshape(C, 64),
      tgt.reshape(C, 1))
    return out[0, 0]


# speedup vs baseline: 17.9260x; 1.7187x over previous
import functools

import jax
import jax.numpy as jnp
from jax import lax
from jax.experimental import pallas as pl
from jax.experimental.pallas import tpu as pltpu
from jax.experimental.pallas import tpu_sc as plsc

C = 4096
L = 16
NCHUNK = C // L
NC = 2
NS = 16
NW = NC * NS
RPW = C // NW
LABEL_SMOOTH = 0.1
EPS = LABEL_SMOOTH / (C - 1)
HI = 1.0 - LABEL_SMOOTH


def _sc_body(preds_hbm, tgt_hbm, stats_hbm, cval_hbm, cidx_hbm,
             rowbuf0, rowbuf1, tgtbuf, stats_v, cval_v, cidx_v, sem0, sem1):
    wid = lax.axis_index("s") * NC + lax.axis_index("c")
    base = wid * RPW
    pltpu.sync_copy(tgt_hbm.at[pl.ds(base, RPW)], tgtbuf)

    iota = lax.iota(jnp.int32, L)
    zf = jnp.zeros((L,), jnp.float32)
    zi = jnp.zeros((L,), jnp.int32)
    ninf = jnp.full((L,), -jnp.inf, jnp.float32)

    bufs = (rowbuf0, rowbuf1)
    sems = (sem0, sem1)
    pltpu.async_copy(preds_hbm.at[pl.ds(base * C, C)], rowbuf0, sem0)

    def do_row(j, rowbuf, sem, osem, obuf):
        row = base + j
        pltpu.make_async_copy(preds_hbm.at[pl.ds(base * C, C)], rowbuf,
                              sem).wait()

        @pl.when(j + 1 < RPW)
        def _():
            pltpu.async_copy(preds_hbm.at[pl.ds((row + 1) * C, C)], obuf,
                             osem)

        def p1(c, carry):
            s, cols, r0v, r1v, r2v, r3v, r0i, r1i, r2i, r3i = carry
            v = rowbuf[pl.ds(c * L, L)]
            s = s + v
            w0 = v > r0v
            w1 = v > r1v
            w2 = v > r2v
            w3 = v > r3v
            n0v = jnp.where(w0, v, r0v)
            n0i = jnp.where(w0, cols, r0i)
            n1v = jnp.where(w0, r0v, jnp.where(w1, v, r1v))
            n1i = jnp.where(w0, r0i, jnp.where(w1, cols, r1i))
            n2v = jnp.where(w1, r1v, jnp.where(w2, v, r2v))
            n2i = jnp.where(w1, r1i, jnp.where(w2, cols, r2i))
            n3v = jnp.where(w2, r2v, jnp.where(w3, v, r3v))
            n3i = jnp.where(w2, r2i, jnp.where(w3, cols, r3i))
            return (s, cols + L, n0v, n1v, n2v, n3v, n0i, n1i, n2i, n3i)

        s, _, r0v, r1v, r2v, r3v, r0i, r1i, r2i, r3i = lax.fori_loop(
            0, NCHUNK, p1,
            (zf, iota, ninf, ninf, ninf, ninf, zi, zi, zi, zi),
            unroll=4)
        m_v = r0v

        def p2(c, e):
            v = rowbuf[pl.ds(c * L, L)]
            return e + jnp.exp(v - m_v)

        e_v = lax.fori_loop(0, NCHUNK, p2, zf, unroll=8)

        tj = plsc.load_gather(tgtbuf, [jnp.full((L,), j, jnp.int32)])
        ptv = plsc.load_gather(rowbuf, [tj])

        sb = j * 64
        stats_v[pl.ds(sb, L)] = m_v
        stats_v[pl.ds(sb + 16, L)] = s
        stats_v[pl.ds(sb + 32, L)] = e_v
        stats_v[pl.ds(sb + 48, L)] = ptv
        cval_v[pl.ds(sb, L)] = r0v
        cval_v[pl.ds(sb + 16, L)] = r1v
        cval_v[pl.ds(sb + 32, L)] = r2v
        cval_v[pl.ds(sb + 48, L)] = r3v
        cidx_v[pl.ds(sb, L)] = r0i
        cidx_v[pl.ds(sb + 16, L)] = r1i
        cidx_v[pl.ds(sb + 32, L)] = r2i
        cidx_v[pl.ds(sb + 48, L)] = r3i

    def pair_step(g, _):
        do_row(2 * g, bufs[0], sems[0], sems[1], bufs[1])
        do_row(2 * g + 1, bufs[1], sems[1], sems[0], bufs[0])
        return 0

    lax.fori_loop(0, RPW // 2, pair_step, 0)

    pltpu.sync_copy(stats_v, stats_hbm.at[pl.ds(base * 64, RPW * 64)])
    pltpu.sync_copy(cval_v, cval_hbm.at[pl.ds(base * 64, RPW * 64)])
    pltpu.sync_copy(cidx_v, cidx_hbm.at[pl.ds(base * 64, RPW * 64)])


def _fin_body(stats_ref, cval_ref, cidx_ref, tgt_ref, out_ref):
    i = pl.program_id(0)
    R = stats_ref.shape[0]
    stats = stats_ref[...]
    m_v = stats[:, 0:16]
    sum_v = stats[:, 16:32]
    e_v = stats[:, 32:48]
    pt = jnp.max(stats[:, 48:64], axis=1)

    M = jnp.max(m_v, axis=1)
    S = jnp.sum(e_v * jnp.exp(m_v - M[:, None]), axis=1)
    lse = M + jnp.log(S)
    rowsum = jnp.sum(sum_v, axis=1)
    full = EPS * (rowsum - C * lse) + (HI - EPS) * (pt - lse)

    cval = cval_ref[...]
    cidx = cidx_ref[...]
    alive = jnp.ones(cval.shape, jnp.bool_)
    tv = []
    ti = []
    for _ in range(4):
        mv = jnp.where(alive, cval, -jnp.inf)
        cur = jnp.max(mv, axis=1)
        cand = mv == cur[:, None]
        curi = jnp.min(jnp.where(cand, cidx, C), axis=1)
        tv.append(cur)
        ti.append(curi)
        alive = alive & ~(cand & (cidx == curi[:, None]))

    rows = i * R + lax.broadcasted_iota(jnp.int32, (R,), 0)
    in0 = ti[0] == rows
    in1 = ti[1] == rows
    in2 = ti[2] == rows
    tgt = tgt_ref[:, 0]

    def term(v, idx):
        w = jnp.where(idx == tgt, HI, EPS)
        return w * (v - lse)

    sk0 = jnp.where(in0, term(tv[1], ti[1]), term(tv[0], ti[0]))
    sk1 = jnp.where(in0 | in1, term(tv[2], ti[2]), term(tv[1], ti[1]))
    sk2 = jnp.where(in0 | in1 | in2, term(tv[3], ti[3]), term(tv[2], ti[2]))
    skipped = sk0 + sk1 + sk2

    loss = -(full - skipped)
    part = jnp.reshape(jnp.sum(loss) * (1.0 / C), (1, 1))

    @pl.when(i == 0)
    def _():
        out_ref[...] = jnp.zeros((1, 1), jnp.float32)

    out_ref[...] += part


def _sc_call(preds_flat, targets):
    mesh = plsc.VectorSubcoreMesh(core_axis_name="c", subcore_axis_name="s",
                                  num_cores=NC, num_subcores=NS)
    f = functools.partial(
        pl.kernel,
        mesh=mesh,
        out_type=[
            jax.ShapeDtypeStruct((C * 64,), jnp.float32),
            jax.ShapeDtypeStruct((C * 64,), jnp.float32),
            jax.ShapeDtypeStruct((C * 64,), jnp.int32),
        ],
        scratch_types=[
            pltpu.VMEM((C,), jnp.float32),
            pltpu.VMEM((C,), jnp.float32),
            pltpu.VMEM((RPW,), jnp.int32),
            pltpu.VMEM((RPW * 64,), jnp.float32),
            pltpu.VMEM((RPW * 64,), jnp.float32),
            pltpu.VMEM((RPW * 64,), jnp.int32),
            pltpu.SemaphoreType.DMA,
            pltpu.SemaphoreType.DMA,
        ],
        compiler_params=pltpu.CompilerParams(needs_layout_passes=False),
    )(_sc_body)
    return f(preds_flat, targets)


def kernel(preds, targets):
    preds_flat = preds.reshape(-1)
    tgt = targets.astype(jnp.int32)
    stats, cval, cidx = _sc_call(preds_flat, tgt)

    R = 512
    out = pl.pallas_call(
        _fin_body,
        grid=(C // R,),
        in_specs=[
            pl.BlockSpec((R, 64), lambda i: (i, 0)),
            pl.BlockSpec((R, 64), lambda i: (i, 0)),
            pl.BlockSpec((R, 64), lambda i: (i, 0)),
            pl.BlockSpec((R, 1), lambda i: (i, 0)),
        ],
        out_specs=pl.BlockSpec((1, 1), lambda i: (0, 0)),
        out_shape=jax.ShapeDtypeStruct((1, 1), jnp.float32),
    )(stats.reshape(C, 64), cval.reshape(C, 64), cidx.reshape(C, 64),
      tgt.reshape(C, 1))
    return out[0, 0]
